# baseline (device time: 23664 ns/iter reference)
import jax
import jax.numpy as jnp
from jax import lax
from jax.experimental import pallas as pl
from jax.experimental.pallas import tpu as pltpu

CHUNKS = [32, 64, 96, 96, 96, 64, 40, 24]
K = len(CHUNKS)


def kernel(x):
    m, n = x.shape
    nh = n // 2
    half = m // 2
    assert sum(CHUNKS) == half
    offs = [sum(CHUNKS[:k]) for k in range(K)]

    def body(x_ref, out_ref, ysend_sems, yrecv_sems, xsend_sems, xrecv_sems,
             copy_sem):
        my_x = lax.axis_index("x")
        my_y = lax.axis_index("y")
        my_z = lax.axis_index("z")
        other_y = 1 - my_y
        other_x = 1 - my_x
        ypeer = (my_x, other_y, my_z)
        xpeer = (other_x, my_y, my_z)

        barrier = pltpu.get_barrier_semaphore()
        for p in (ypeer, xpeer):
            pl.semaphore_signal(
                barrier, inc=1, device_id=p,
                device_id_type=pl.DeviceIdType.MESH,
            )
        pl.semaphore_wait(barrier, 2)

        src_row0 = my_x * half
        dst_row0 = my_y * m + my_x * half
        fwd_row0 = other_y * m + my_x * half

        y_rdmas = []
        for k in range(K):
            o, blk = offs[k], CHUNKS[k]
            r = pltpu.make_async_remote_copy(
                src_ref=x_ref.at[
                    pl.ds(src_row0 + o, blk), pl.ds(other_y * nh, nh)
                ],
                dst_ref=out_ref.at[pl.ds(dst_row0 + o, blk), :],
                send_sem=ysend_sems.at[k],
                recv_sem=yrecv_sems.at[k],
                device_id=ypeer,
                device_id_type=pl.DeviceIdType.MESH,
            )
            r.start()
            y_rdmas.append(r)

        local_copy = pltpu.make_async_copy(
            x_ref.at[:, pl.ds(my_y * nh, nh)],
            out_ref.at[pl.ds(my_y * m, m), :],
            copy_sem,
        )
        local_copy.start()

        x_rdmas = []
        for k in range(K):
            o, blk = offs[k], CHUNKS[k]
            y_rdmas[k].wait_recv()
            r = pltpu.make_async_remote_copy(
                src_ref=out_ref.at[pl.ds(fwd_row0 + o, blk), :],
                dst_ref=out_ref.at[pl.ds(fwd_row0 + o, blk), :],
                send_sem=xsend_sems.at[k],
                recv_sem=xrecv_sems.at[k],
                device_id=xpeer,
                device_id_type=pl.DeviceIdType.MESH,
            )
            r.start()
            x_rdmas.append(r)

        for k in range(K):
            y_rdmas[k].wait_send()
            x_rdmas[k].wait_send()
            x_rdmas[k].wait_recv()
        local_copy.wait()

    return pl.pallas_call(
        body,
        out_shape=jax.ShapeDtypeStruct((2 * m, nh), x.dtype),
        in_specs=[pl.BlockSpec(memory_space=pl.MemorySpace.ANY)],
        out_specs=pl.BlockSpec(memory_space=pl.MemorySpace.ANY),
        scratch_shapes=[
            pltpu.SemaphoreType.DMA((K,)),
            pltpu.SemaphoreType.DMA((K,)),
            pltpu.SemaphoreType.DMA((K,)),
            pltpu.SemaphoreType.DMA((K,)),
            pltpu.SemaphoreType.DMA,
        ],
        compiler_params=pltpu.CompilerParams(collective_id=0),
    )(x)


# device time: 22974 ns/iter; 1.0300x vs baseline; 1.0300x over previous
import jax
import jax.numpy as jnp
from jax import lax
from jax.experimental import pallas as pl
from jax.experimental.pallas import tpu as pltpu

CHUNKS = [64, 64, 64, 64, 64, 64, 64, 64]
K = len(CHUNKS)


def kernel(x):
    m, n = x.shape
    nh = n // 2
    half = m // 2
    assert sum(CHUNKS) == half
    offs = [sum(CHUNKS[:k]) for k in range(K)]

    def body(x_ref, out_ref, ysend_sems, yrecv_sems, xsend_sems, xrecv_sems,
             copy_sem):
        my_x = lax.axis_index("x")
        my_y = lax.axis_index("y")
        my_z = lax.axis_index("z")
        other_y = 1 - my_y
        other_x = 1 - my_x
        ypeer = (my_x, other_y, my_z)
        xpeer = (other_x, my_y, my_z)

        barrier = pltpu.get_barrier_semaphore()
        for p in (ypeer, xpeer):
            pl.semaphore_signal(
                barrier, inc=1, device_id=p,
                device_id_type=pl.DeviceIdType.MESH,
            )
        pl.semaphore_wait(barrier, 2)

        src_row0 = my_x * half
        dst_row0 = my_y * m + my_x * half
        fwd_row0 = other_y * m + my_x * half

        y_rdmas = []
        for k in range(K):
            o, blk = offs[k], CHUNKS[k]
            r = pltpu.make_async_remote_copy(
                src_ref=x_ref.at[
                    pl.ds(src_row0 + o, blk), pl.ds(other_y * nh, nh)
                ],
                dst_ref=out_ref.at[pl.ds(dst_row0 + o, blk), :],
                send_sem=ysend_sems.at[k],
                recv_sem=yrecv_sems.at[k],
                device_id=ypeer,
                device_id_type=pl.DeviceIdType.MESH,
            )
            r.start()
            y_rdmas.append(r)

        local_copy = pltpu.make_async_copy(
            x_ref.at[:, pl.ds(my_y * nh, nh)],
            out_ref.at[pl.ds(my_y * m, m), :],
            copy_sem,
        )
        local_copy.start()

        x_rdmas = []
        for k in range(K):
            o, blk = offs[k], CHUNKS[k]
            y_rdmas[k].wait_recv()
            r = pltpu.make_async_remote_copy(
                src_ref=out_ref.at[pl.ds(fwd_row0 + o, blk), :],
                dst_ref=out_ref.at[pl.ds(fwd_row0 + o, blk), :],
                send_sem=xsend_sems.at[k],
                recv_sem=xrecv_sems.at[k],
                device_id=xpeer,
                device_id_type=pl.DeviceIdType.MESH,
            )
            r.start()
            x_rdmas.append(r)

        for k in range(K):
            y_rdmas[k].wait_send()
            x_rdmas[k].wait_send()
            x_rdmas[k].wait_recv()
        local_copy.wait()

    return pl.pallas_call(
        body,
        out_shape=jax.ShapeDtypeStruct((2 * m, nh), x.dtype),
        in_specs=[pl.BlockSpec(memory_space=pl.MemorySpace.ANY)],
        out_specs=pl.BlockSpec(memory_space=pl.MemorySpace.ANY),
        scratch_shapes=[
            pltpu.SemaphoreType.DMA((K,)),
            pltpu.SemaphoreType.DMA((K,)),
            pltpu.SemaphoreType.DMA((K,)),
            pltpu.SemaphoreType.DMA((K,)),
            pltpu.SemaphoreType.DMA,
        ],
        compiler_params=pltpu.CompilerParams(collective_id=0),
    )(x)


# device time: 21186 ns/iter; 1.1170x vs baseline; 1.0844x over previous
import jax
import jax.numpy as jnp
from jax import lax
from jax.experimental import pallas as pl
from jax.experimental.pallas import tpu as pltpu

C = 4


def kernel(x):
    m, n = x.shape
    nh = n // 2
    q_rows = m // 4
    blk = q_rows // C
    hblk = q_rows // 2

    def body(x_ref, out_ref, ysend, yrecv, xqsend, xqrecv, zqsend, zqrecv,
             fasend, farecv, fbsend, fbrecv, copy_sem):
        my_x = lax.axis_index("x")
        my_y = lax.axis_index("y")
        my_z = lax.axis_index("z")
        other_y = 1 - my_y
        other_x = 1 - my_x
        pz = my_z % 2
        zp_z = my_z + 1 - 2 * pz
        ypeer = (my_x, other_y, my_z)
        xpeer = (other_x, my_y, my_z)
        zpeer = (my_x, my_y, zp_z)

        barrier = pltpu.get_barrier_semaphore()
        for p in (ypeer, xpeer, zpeer):
            pl.semaphore_signal(
                barrier, inc=1, device_id=p,
                device_id_type=pl.DeviceIdType.MESH,
            )
        pl.semaphore_wait(barrier, 3)

        q_me = 2 * my_x + pz
        q_xp = 2 * other_x + pz
        q_zp = 2 * my_x + (1 - pz)
        rem0 = other_y * m
        myq0 = rem0 + q_rows * q_me
        xq0 = rem0 + q_rows * q_xp
        zq0 = rem0 + q_rows * q_zp

        y_rdmas = []
        for c in range(C):
            r = pltpu.make_async_remote_copy(
                src_ref=x_ref.at[
                    pl.ds(q_rows * q_me + c * blk, blk),
                    pl.ds(other_y * nh, nh),
                ],
                dst_ref=out_ref.at[
                    pl.ds(my_y * m + q_rows * q_me + c * blk, blk), :
                ],
                send_sem=ysend.at[c],
                recv_sem=yrecv.at[c],
                device_id=ypeer,
                device_id_type=pl.DeviceIdType.MESH,
            )
            r.start()
            y_rdmas.append(r)

        local_copy = pltpu.make_async_copy(
            x_ref.at[:, pl.ds(my_y * nh, nh)],
            out_ref.at[pl.ds(my_y * m, m), :],
            copy_sem,
        )
        local_copy.start()

        xq_rdmas = []
        zq_rdmas = []
        for c in range(C):
            y_rdmas[c].wait_recv()
            src = out_ref.at[pl.ds(myq0 + c * blk, blk), :]
            r = pltpu.make_async_remote_copy(
                src_ref=src,
                dst_ref=out_ref.at[pl.ds(myq0 + c * blk, blk), :],
                send_sem=xqsend.at[c],
                recv_sem=xqrecv.at[c],
                device_id=xpeer,
                device_id_type=pl.DeviceIdType.MESH,
            )
            r.start()
            xq_rdmas.append(r)
            r = pltpu.make_async_remote_copy(
                src_ref=src,
                dst_ref=out_ref.at[pl.ds(myq0 + c * blk, blk), :],
                send_sem=zqsend.at[c],
                recv_sem=zqrecv.at[c],
                device_id=zpeer,
                device_id_type=pl.DeviceIdType.MESH,
            )
            r.start()
            zq_rdmas.append(r)

        fa_rdmas = []
        for c in range(C // 2):
            xq_rdmas[c].wait_recv()
            r = pltpu.make_async_remote_copy(
                src_ref=out_ref.at[pl.ds(xq0 + c * blk, blk), :],
                dst_ref=out_ref.at[pl.ds(xq0 + c * blk, blk), :],
                send_sem=fasend.at[c],
                recv_sem=farecv.at[c],
                device_id=zpeer,
                device_id_type=pl.DeviceIdType.MESH,
            )
            r.start()
            fa_rdmas.append(r)

        fb_rdmas = []
        for c in range(C // 2):
            zq_rdmas[C // 2 + c].wait_recv()
            r = pltpu.make_async_remote_copy(
                src_ref=out_ref.at[pl.ds(zq0 + hblk + c * blk, blk), :],
                dst_ref=out_ref.at[pl.ds(zq0 + hblk + c * blk, blk), :],
                send_sem=fbsend.at[c],
                recv_sem=fbrecv.at[c],
                device_id=xpeer,
                device_id_type=pl.DeviceIdType.MESH,
            )
            r.start()
            fb_rdmas.append(r)

        for c in range(C // 2, C):
            xq_rdmas[c].wait_recv()
        for c in range(C // 2):
            zq_rdmas[c].wait_recv()
        for c in range(C // 2):
            fa_rdmas[c].wait_recv()
            fb_rdmas[c].wait_recv()
        for c in range(C):
            y_rdmas[c].wait_send()
            xq_rdmas[c].wait_send()
            zq_rdmas[c].wait_send()
        for c in range(C // 2):
            fa_rdmas[c].wait_send()
            fb_rdmas[c].wait_send()
        local_copy.wait()

    return pl.pallas_call(
        body,
        out_shape=jax.ShapeDtypeStruct((2 * m, nh), x.dtype),
        in_specs=[pl.BlockSpec(memory_space=pl.MemorySpace.ANY)],
        out_specs=pl.BlockSpec(memory_space=pl.MemorySpace.ANY),
        scratch_shapes=[
            pltpu.SemaphoreType.DMA((C,)),
            pltpu.SemaphoreType.DMA((C,)),
            pltpu.SemaphoreType.DMA((C,)),
            pltpu.SemaphoreType.DMA((C,)),
            pltpu.SemaphoreType.DMA((C,)),
            pltpu.SemaphoreType.DMA((C,)),
            pltpu.SemaphoreType.DMA((C // 2,)),
            pltpu.SemaphoreType.DMA((C // 2,)),
            pltpu.SemaphoreType.DMA((C // 2,)),
            pltpu.SemaphoreType.DMA((C // 2,)),
            pltpu.SemaphoreType.DMA,
        ],
        compiler_params=pltpu.CompilerParams(collective_id=0),
    )(x)


# device time: 20040 ns/iter; 1.1808x vs baseline; 1.0572x over previous
import jax
import jax.numpy as jnp
from jax import lax
from jax.experimental import pallas as pl
from jax.experimental.pallas import tpu as pltpu

C = 4
Q_ROWS = 224
T_ROWS = 128
TC = 2


def kernel(x):
    m, n = x.shape
    nh = n // 2
    q_rows = Q_ROWS
    blk = q_rows // C
    hblk = q_rows // 2
    t0_row = 4 * q_rows
    tblk = T_ROWS // TC
    assert 4 * q_rows + T_ROWS == m

    def body(x_ref, out_ref, ysend, yrecv, tsend, trecv, xqsend, xqrecv,
             zqsend, zqrecv, fasend, farecv, fbsend, fbrecv, copy_sem):
        my_x = lax.axis_index("x")
        my_y = lax.axis_index("y")
        my_z = lax.axis_index("z")
        other_y = 1 - my_y
        other_x = 1 - my_x
        pz = my_z % 2
        zp_z = my_z + 1 - 2 * pz
        ypeer = (my_x, other_y, my_z)
        xpeer = (other_x, my_y, my_z)
        zpeer = (my_x, my_y, zp_z)

        barrier = pltpu.get_barrier_semaphore()
        for p in (ypeer, xpeer, zpeer):
            pl.semaphore_signal(
                barrier, inc=1, device_id=p,
                device_id_type=pl.DeviceIdType.MESH,
            )
        pl.semaphore_wait(barrier, 3)

        q_me = 2 * my_x + pz
        q_xp = 2 * other_x + pz
        q_zp = 2 * my_x + (1 - pz)
        rem0 = other_y * m
        myq0 = rem0 + q_rows * q_me
        xq0 = rem0 + q_rows * q_xp
        zq0 = rem0 + q_rows * q_zp

        y_rdmas = []
        for c in range(C):
            r = pltpu.make_async_remote_copy(
                src_ref=x_ref.at[
                    pl.ds(q_rows * q_me + c * blk, blk),
                    pl.ds(other_y * nh, nh),
                ],
                dst_ref=out_ref.at[
                    pl.ds(my_y * m + q_rows * q_me + c * blk, blk), :
                ],
                send_sem=ysend.at[c],
                recv_sem=yrecv.at[c],
                device_id=ypeer,
                device_id_type=pl.DeviceIdType.MESH,
            )
            r.start()
            y_rdmas.append(r)

        t_rdmas = []
        for c in range(TC):
            r = pltpu.make_async_remote_copy(
                src_ref=x_ref.at[
                    pl.ds(t0_row + c * tblk, tblk), pl.ds(other_y * nh, nh)
                ],
                dst_ref=out_ref.at[
                    pl.ds(my_y * m + t0_row + c * tblk, tblk), :
                ],
                send_sem=tsend.at[c],
                recv_sem=trecv.at[c],
                device_id=ypeer,
                device_id_type=pl.DeviceIdType.MESH,
            )
            r.start()
            t_rdmas.append(r)

        local_copy = pltpu.make_async_copy(
            x_ref.at[:, pl.ds(my_y * nh, nh)],
            out_ref.at[pl.ds(my_y * m, m), :],
            copy_sem,
        )
        local_copy.start()

        xq_rdmas = []
        zq_rdmas = []
        for c in range(C):
            y_rdmas[c].wait_recv()
            src = out_ref.at[pl.ds(myq0 + c * blk, blk), :]
            r = pltpu.make_async_remote_copy(
                src_ref=src,
                dst_ref=out_ref.at[pl.ds(myq0 + c * blk, blk), :],
                send_sem=xqsend.at[c],
                recv_sem=xqrecv.at[c],
                device_id=xpeer,
                device_id_type=pl.DeviceIdType.MESH,
            )
            r.start()
            xq_rdmas.append(r)
            r = pltpu.make_async_remote_copy(
                src_ref=src,
                dst_ref=out_ref.at[pl.ds(myq0 + c * blk, blk), :],
                send_sem=zqsend.at[c],
                recv_sem=zqrecv.at[c],
                device_id=zpeer,
                device_id_type=pl.DeviceIdType.MESH,
            )
            r.start()
            zq_rdmas.append(r)

        fa_rdmas = []
        for c in range(C // 2):
            xq_rdmas[c].wait_recv()
            r = pltpu.make_async_remote_copy(
                src_ref=out_ref.at[pl.ds(xq0 + c * blk, blk), :],
                dst_ref=out_ref.at[pl.ds(xq0 + c * blk, blk), :],
                send_sem=fasend.at[c],
                recv_sem=farecv.at[c],
                device_id=zpeer,
                device_id_type=pl.DeviceIdType.MESH,
            )
            r.start()
            fa_rdmas.append(r)

        fb_rdmas = []
        for c in range(C // 2):
            zq_rdmas[C // 2 + c].wait_recv()
            r = pltpu.make_async_remote_copy(
                src_ref=out_ref.at[pl.ds(zq0 + hblk + c * blk, blk), :],
                dst_ref=out_ref.at[pl.ds(zq0 + hblk + c * blk, blk), :],
                send_sem=fbsend.at[c],
                recv_sem=fbrecv.at[c],
                device_id=xpeer,
                device_id_type=pl.DeviceIdType.MESH,
            )
            r.start()
            fb_rdmas.append(r)

        for c in range(C // 2, C):
            xq_rdmas[c].wait_recv()
        for c in range(C // 2):
            zq_rdmas[c].wait_recv()
        for c in range(C // 2):
            fa_rdmas[c].wait_recv()
            fb_rdmas[c].wait_recv()
        for c in range(TC):
            t_rdmas[c].wait_recv()
            t_rdmas[c].wait_send()
        for c in range(C):
            y_rdmas[c].wait_send()
            xq_rdmas[c].wait_send()
            zq_rdmas[c].wait_send()
        for c in range(C // 2):
            fa_rdmas[c].wait_send()
            fb_rdmas[c].wait_send()
        local_copy.wait()

    return pl.pallas_call(
        body,
        out_shape=jax.ShapeDtypeStruct((2 * m, nh), x.dtype),
        in_specs=[pl.BlockSpec(memory_space=pl.MemorySpace.ANY)],
        out_specs=pl.BlockSpec(memory_space=pl.MemorySpace.ANY),
        scratch_shapes=[
            pltpu.SemaphoreType.DMA((C,)),
            pltpu.SemaphoreType.DMA((C,)),
            pltpu.SemaphoreType.DMA((TC,)),
            pltpu.SemaphoreType.DMA((TC,)),
            pltpu.SemaphoreType.DMA((C,)),
            pltpu.SemaphoreType.DMA((C,)),
            pltpu.SemaphoreType.DMA((C,)),
            pltpu.SemaphoreType.DMA((C,)),
            pltpu.SemaphoreType.DMA((C // 2,)),
            pltpu.SemaphoreType.DMA((C // 2,)),
            pltpu.SemaphoreType.DMA((C // 2,)),
            pltpu.SemaphoreType.DMA((C // 2,)),
            pltpu.SemaphoreType.DMA,
        ],
        compiler_params=pltpu.CompilerParams(collective_id=0),
    )(x)
